# Initial kernel scaffold; baseline (speedup 1.0000x reference)
import jax
import jax.numpy as jnp
from jax.experimental import pallas as pl


def kernel(x, w):
    raise NotImplementedError("write your pallas kernel here")


# speedup vs baseline: 2.2376x; 2.2376x over previous
import functools

import jax
import jax.numpy as jnp
from jax.experimental import pallas as pl
from jax.experimental.pallas import tpu as pltpu


def _conv3x3_kernel(x_ref, w_ref, o_ref, buf_ref, *, W, L, Mg):
    C = x_ref.shape[1]
    bf16 = jnp.bfloat16
    buf_ref[:, :Mg] = jnp.zeros((C, Mg), bf16)
    buf_ref[:, Mg + L:] = jnp.zeros((C, Mg), bf16)
    buf_ref[:, Mg:Mg + L] = x_ref[0].astype(bf16)

    col = jax.lax.broadcasted_iota(jnp.int32, (1, L), 1) % W
    not_first = (col != 0).astype(jnp.float32)
    not_last = (col != W - 1).astype(jnp.float32)

    def tap(kh, kw):
        off = Mg + (kh - 1) * W + (kw - 1)
        return jnp.dot(w_ref[kh * 3 + kw], buf_ref[:, off:off + L],
                       preferred_element_type=jnp.float32)

    left = tap(0, 0) + tap(1, 0) + tap(2, 0)
    mid = tap(0, 1) + tap(1, 1) + tap(2, 1)
    right = tap(0, 2) + tap(1, 2) + tap(2, 2)
    o_ref[0] = mid + left * not_first + right * not_last


def kernel(x, w):
    N, Cin, H, W = x.shape
    Cout, _, K, _ = w.shape
    assert K == 3
    L = H * W
    Mg = 128

    x_flat = x.reshape(N, Cin, L)
    wt = jnp.transpose(w, (2, 3, 0, 1)).reshape(
        K * K, Cout, Cin).astype(jnp.bfloat16)

    out = pl.pallas_call(
        functools.partial(_conv3x3_kernel, W=W, L=L, Mg=Mg),
        out_shape=jax.ShapeDtypeStruct((N, Cout, L), jnp.float32),
        grid=(N,),
        in_specs=[
            pl.BlockSpec((1, Cin, L), lambda n: (n, 0, 0)),
            pl.BlockSpec((K * K, Cout, Cin), lambda n: (0, 0, 0)),
        ],
        out_specs=pl.BlockSpec((1, Cout, L), lambda n: (n, 0, 0)),
        scratch_shapes=[pltpu.VMEM((Cin, 2 * Mg + L), jnp.bfloat16)],
        compiler_params=pltpu.CompilerParams(
            dimension_semantics=("parallel",)),
    )(x_flat, wt)
    return out.reshape(N, Cout, H, W)
